# baseline (speedup 1.0000x reference)
import jax
import jax.numpy as jnp
from jax.experimental import pallas as pl


def _chamfer_body(p1_ref, p2t_ref, out_ref):
    p1 = p1_ref[0]
    p2t = p2t_ref[0]
    c = jax.lax.dot_general(
        p1, p2t, (((1,), (0,)), ((), ())),
        preferred_element_type=jnp.float32,
        precision=jax.lax.Precision.HIGHEST,
    )
    a = jnp.sum(p1 * p1, axis=1, keepdims=True)
    b = jnp.sum(p2t * p2t, axis=0, keepdims=True) * 0.25
    t1 = b - c
    rmin = a[:, 0] + jnp.min(t1, axis=1)
    t2 = a - c
    cmin = b[0, :] + jnp.min(t2, axis=0)
    d_fwd = jnp.mean(jnp.sqrt(jnp.maximum(rmin, 0.0)))
    d_bwd = jnp.mean(jnp.sqrt(jnp.maximum(cmin, 0.0)))
    out_ref[...] = ((d_fwd + d_bwd) * 0.5).reshape(1, 1, 1)


def kernel(point_set_1, point_set_2):
    b, p1, _ = point_set_1.shape
    p2 = point_set_2.shape[1]
    p2t = jnp.transpose(point_set_2, (0, 2, 1)) * 2.0
    out = pl.pallas_call(
        _chamfer_body,
        grid=(b,),
        in_specs=[
            pl.BlockSpec((1, p1, 2), lambda i: (i, 0, 0)),
            pl.BlockSpec((1, 2, p2), lambda i: (i, 0, 0)),
        ],
        out_specs=pl.BlockSpec((1, 1, 1), lambda i: (i, 0, 0)),
        out_shape=jax.ShapeDtypeStruct((b, 1, 1), jnp.float32),
    )(point_set_1, p2t)
    return out[:, 0, 0]


# speedup vs baseline: 2.9493x; 2.9493x over previous
import jax
import jax.numpy as jnp
from jax.experimental import pallas as pl

_BATCH_TILE = 4


def _chamfer_body(x1_ref, y1_ref, x2_ref, y2_ref, out_ref):
    vals = []
    for t in range(_BATCH_TILE):
        x1 = x1_ref[t, 0, :]
        y1 = y1_ref[t, 0, :]
        x2 = x2_ref[t, 0, :]
        y2 = y2_ref[t, 0, :]
        dx = x1[:, None] - x2[None, :]
        dy = y1[:, None] - y2[None, :]
        d2 = dx * dx + dy * dy
        rmin = jnp.min(d2, axis=1)
        cmin = jnp.min(d2, axis=0)
        d_fwd = jnp.mean(jnp.sqrt(rmin))
        d_bwd = jnp.mean(jnp.sqrt(cmin))
        vals.append((d_fwd + d_bwd) * 0.5)
    out_ref[...] = jnp.stack(vals).reshape(_BATCH_TILE, 1, 1)


def kernel(point_set_1, point_set_2):
    b, p1, _ = point_set_1.shape
    p2 = point_set_2.shape[1]
    t = _BATCH_TILE
    x1 = point_set_1[:, :, 0].reshape(b, 1, p1)
    y1 = point_set_1[:, :, 1].reshape(b, 1, p1)
    x2 = point_set_2[:, :, 0].reshape(b, 1, p2)
    y2 = point_set_2[:, :, 1].reshape(b, 1, p2)
    out = pl.pallas_call(
        _chamfer_body,
        grid=(b // t,),
        in_specs=[
            pl.BlockSpec((t, 1, p1), lambda i: (i, 0, 0)),
            pl.BlockSpec((t, 1, p1), lambda i: (i, 0, 0)),
            pl.BlockSpec((t, 1, p2), lambda i: (i, 0, 0)),
            pl.BlockSpec((t, 1, p2), lambda i: (i, 0, 0)),
        ],
        out_specs=pl.BlockSpec((t, 1, 1), lambda i: (i, 0, 0)),
        out_shape=jax.ShapeDtypeStruct((b, 1, 1), jnp.float32),
    )(x1, y1, x2, y2)
    return out[:, 0, 0]


# speedup vs baseline: 3.0539x; 1.0355x over previous
import jax
import jax.numpy as jnp
from jax.experimental import pallas as pl

_BATCH_TILE = 8


def _chamfer_body(x1_ref, y1_ref, x2_ref, y2_ref, out_ref):
    vals = []
    for t in range(_BATCH_TILE):
        x1 = x1_ref[t, 0, :]
        y1 = y1_ref[t, 0, :]
        x2 = x2_ref[t, 0, :]
        y2 = y2_ref[t, 0, :]
        dx = x1[:, None] - x2[None, :]
        dy = y1[:, None] - y2[None, :]
        d2 = dx * dx + dy * dy
        rmin = jnp.min(d2, axis=1)
        cmin = jnp.min(d2, axis=0)
        d_fwd = jnp.mean(jnp.sqrt(rmin))
        d_bwd = jnp.mean(jnp.sqrt(cmin))
        vals.append((d_fwd + d_bwd) * 0.5)
    out_ref[...] = jnp.stack(vals).reshape(_BATCH_TILE, 1, 1)


def kernel(point_set_1, point_set_2):
    b, p1, _ = point_set_1.shape
    p2 = point_set_2.shape[1]
    t = _BATCH_TILE
    x1 = point_set_1[:, :, 0].reshape(b, 1, p1)
    y1 = point_set_1[:, :, 1].reshape(b, 1, p1)
    x2 = point_set_2[:, :, 0].reshape(b, 1, p2)
    y2 = point_set_2[:, :, 1].reshape(b, 1, p2)
    out = pl.pallas_call(
        _chamfer_body,
        grid=(b // t,),
        in_specs=[
            pl.BlockSpec((t, 1, p1), lambda i: (i, 0, 0)),
            pl.BlockSpec((t, 1, p1), lambda i: (i, 0, 0)),
            pl.BlockSpec((t, 1, p2), lambda i: (i, 0, 0)),
            pl.BlockSpec((t, 1, p2), lambda i: (i, 0, 0)),
        ],
        out_specs=pl.BlockSpec((t, 1, 1), lambda i: (i, 0, 0)),
        out_shape=jax.ShapeDtypeStruct((b, 1, 1), jnp.float32),
    )(x1, y1, x2, y2)
    return out[:, 0, 0]
